# Initial kernel scaffold; baseline (speedup 1.0000x reference)
import jax
import jax.numpy as jnp
from jax.experimental import pallas as pl


def kernel(x, table):
    raise NotImplementedError("write your pallas kernel here")


# speedup vs baseline: 3.3498x; 3.3498x over previous
import functools

import jax
import jax.numpy as jnp
from jax import lax
from jax.experimental import pallas as pl
from jax.experimental.pallas import tpu as pltpu
from jax.experimental.pallas import tpu_sc as plsc

BATCH = 4096
HIST = 50
EMBED = 128
TOTAL = BATCH * HIST
NUM_WORKERS = 32
PER_WORKER = TOTAL // NUM_WORKERS
CHUNK = 128
NCHUNK = PER_WORKER // CHUNK
NBUF = 4

_mesh = plsc.VectorSubcoreMesh(core_axis_name="c", subcore_axis_name="s")


@functools.partial(
    pl.kernel,
    out_type=jax.ShapeDtypeStruct((TOTAL, EMBED), jnp.float32),
    mesh=_mesh,
    scratch_types=[
        pltpu.VMEM((NCHUNK, CHUNK), jnp.int32),
        pltpu.VMEM((NBUF, CHUNK, EMBED), jnp.float32),
        pltpu.SemaphoreType.DMA,
        pltpu.SemaphoreType.DMA,
    ],
)
def _emb_gather(idx_hbm, table_hbm, out_hbm, idx_v, rows_v, gsem, ssem):
    wid = lax.axis_index("s") * 2 + lax.axis_index("c")
    base = wid * PER_WORKER
    pltpu.sync_copy(idx_hbm.at[wid], idx_v)

    def g_copy(c):
        return pltpu.make_async_copy(
            table_hbm.at[idx_v.at[c]], rows_v.at[c % NBUF], gsem)

    def s_copy(c):
        return pltpu.make_async_copy(
            rows_v.at[c % NBUF],
            out_hbm.at[pl.ds(base + c * CHUNK, CHUNK)], ssem)

    for b in range(NBUF - 1):
        g_copy(b).start()

    @pl.loop(0, NCHUNK)
    def _body(c):
        @pl.when(c > 0)
        def _():
            s_copy(c - 1).wait()

        @pl.when(c + NBUF - 1 < NCHUNK)
        def _():
            g_copy(c + NBUF - 1).start()

        g_copy(c).wait()
        s_copy(c).start()

    s_copy(NCHUNK - 1).wait()


def kernel(x, table):
    idx = x.reshape(TOTAL).astype(jnp.int32).reshape(NUM_WORKERS, NCHUNK, CHUNK)
    out = _emb_gather(idx, table)
    return out.reshape(BATCH, HIST, EMBED)


# speedup vs baseline: 3.3715x; 1.0065x over previous
import functools

import jax
import jax.numpy as jnp
from jax import lax
from jax.experimental import pallas as pl
from jax.experimental.pallas import tpu as pltpu
from jax.experimental.pallas import tpu_sc as plsc

BATCH = 4096
HIST = 50
EMBED = 128
TOTAL = BATCH * HIST
NUM_WORKERS = 32
PER_WORKER = TOTAL // NUM_WORKERS
CHUNK = 128
NCHUNK = PER_WORKER // CHUNK
NBUF = 6

_mesh = plsc.VectorSubcoreMesh(core_axis_name="c", subcore_axis_name="s")


@functools.partial(
    pl.kernel,
    out_type=jax.ShapeDtypeStruct((TOTAL, EMBED), jnp.float32),
    mesh=_mesh,
    scratch_types=[
        pltpu.VMEM((NCHUNK, CHUNK), jnp.int32),
        pltpu.VMEM((NBUF, CHUNK, EMBED), jnp.float32),
        pltpu.SemaphoreType.DMA,
        pltpu.SemaphoreType.DMA,
    ],
)
def _emb_gather(idx_hbm, table_hbm, out_hbm, idx_v, rows_v, gsem, ssem):
    wid = lax.axis_index("s") * 2 + lax.axis_index("c")
    base = wid * PER_WORKER
    pltpu.sync_copy(idx_hbm.at[wid], idx_v)

    def g_copy(c):
        return pltpu.make_async_copy(
            table_hbm.at[idx_v.at[c]], rows_v.at[c % NBUF], gsem)

    def s_copy(c):
        return pltpu.make_async_copy(
            rows_v.at[c % NBUF],
            out_hbm.at[pl.ds(base + c * CHUNK, CHUNK)], ssem)

    for b in range(NBUF - 1):
        g_copy(b).start()

    @pl.loop(0, NCHUNK)
    def _body(c):
        @pl.when(c > 0)
        def _():
            s_copy(c - 1).wait()

        @pl.when(c + NBUF - 1 < NCHUNK)
        def _():
            g_copy(c + NBUF - 1).start()

        g_copy(c).wait()
        s_copy(c).start()

    s_copy(NCHUNK - 1).wait()


def kernel(x, table):
    idx = x.reshape(TOTAL).astype(jnp.int32).reshape(NUM_WORKERS, NCHUNK, CHUNK)
    out = _emb_gather(idx, table)
    return out.reshape(BATCH, HIST, EMBED)


# speedup vs baseline: 6.0219x; 1.7861x over previous
import functools

import jax
import jax.numpy as jnp
from jax import lax
from jax.experimental import pallas as pl
from jax.experimental.pallas import tpu as pltpu
from jax.experimental.pallas import tpu_sc as plsc

BATCH = 4096
HIST = 50
EMBED = 128
NUM_WORKERS = 32
ROWS_PER_W = BATCH // NUM_WORKERS
K = 8
NGROUP = ROWS_PER_W // K
NBUF = 2

_mesh = plsc.VectorSubcoreMesh(core_axis_name="c", subcore_axis_name="s")


@functools.partial(
    pl.kernel,
    out_type=jax.ShapeDtypeStruct((BATCH, HIST, EMBED), jnp.float32),
    mesh=_mesh,
    scratch_types=[
        pltpu.VMEM((ROWS_PER_W, HIST), jnp.int32),
        pltpu.VMEM((NBUF, K, HIST, EMBED), jnp.float32),
        pltpu.SemaphoreType.DMA,
        pltpu.SemaphoreType.DMA,
    ],
)
def _emb_gather(idx_hbm, table_hbm, out_hbm, idx_v, rows_v, gsem, ssem):
    wid = lax.axis_index("s") * 2 + lax.axis_index("c")
    base = wid * ROWS_PER_W
    pltpu.sync_copy(idx_hbm.at[pl.ds(base, ROWS_PER_W)], idx_v)

    def start_group(g):
        for k in range(K):
            pltpu.make_async_copy(
                table_hbm.at[idx_v.at[g * K + k]],
                rows_v.at[g % NBUF, k], gsem).start()

    def wait_group(g):
        for k in range(K):
            pltpu.make_async_copy(
                table_hbm.at[idx_v.at[g * K + k]],
                rows_v.at[g % NBUF, k], gsem).wait()

    def s_copy(g):
        return pltpu.make_async_copy(
            rows_v.at[g % NBUF],
            out_hbm.at[pl.ds(base + g * K, K)], ssem)

    start_group(0)

    @pl.loop(0, NGROUP)
    def _body(g):
        @pl.when(g > 0)
        def _():
            s_copy(g - 1).wait()

        @pl.when(g + 1 < NGROUP)
        def _():
            start_group(g + 1)

        wait_group(g)
        s_copy(g).start()

    s_copy(NGROUP - 1).wait()


def kernel(x, table):
    return _emb_gather(x.astype(jnp.int32), table)


# speedup vs baseline: 6.0228x; 1.0002x over previous
import functools

import jax
import jax.numpy as jnp
from jax import lax
from jax.experimental import pallas as pl
from jax.experimental.pallas import tpu as pltpu
from jax.experimental.pallas import tpu_sc as plsc

BATCH = 4096
HIST = 50
EMBED = 128
NUM_WORKERS = 32
ROWS_PER_W = BATCH // NUM_WORKERS
K = 8
NGROUP = ROWS_PER_W // K
NBUF = 2

_mesh = plsc.VectorSubcoreMesh(core_axis_name="c", subcore_axis_name="s")


@functools.partial(
    pl.kernel,
    out_type=jax.ShapeDtypeStruct((BATCH, HIST, EMBED), jnp.float32),
    mesh=_mesh,
    scratch_types=[
        pltpu.VMEM((ROWS_PER_W, HIST), jnp.int32),
        pltpu.VMEM((NBUF, K, HIST, EMBED), jnp.float32),
        pltpu.SemaphoreType.DMA,
        pltpu.SemaphoreType.DMA,
    ],
    compiler_params=pltpu.CompilerParams(use_tc_tiling_on_sc=True),
)
def _emb_gather(idx_hbm, table_hbm, out_hbm, idx_v, rows_v, gsem, ssem):
    wid = lax.axis_index("s") * 2 + lax.axis_index("c")
    base = wid * ROWS_PER_W
    pltpu.sync_copy(idx_hbm.at[pl.ds(base, ROWS_PER_W)], idx_v)

    def start_group(g):
        for k in range(K):
            pltpu.make_async_copy(
                table_hbm.at[idx_v.at[g * K + k]],
                rows_v.at[g % NBUF, k], gsem).start()

    def wait_group(g):
        for k in range(K):
            pltpu.make_async_copy(
                table_hbm.at[idx_v.at[g * K + k]],
                rows_v.at[g % NBUF, k], gsem).wait()

    def s_copy(g):
        return pltpu.make_async_copy(
            rows_v.at[g % NBUF],
            out_hbm.at[pl.ds(base + g * K, K)], ssem)

    start_group(0)

    @pl.loop(0, NGROUP)
    def _body(g):
        @pl.when(g > 0)
        def _():
            s_copy(g - 1).wait()

        @pl.when(g + 1 < NGROUP)
        def _():
            start_group(g + 1)

        wait_group(g)
        s_copy(g).start()

    s_copy(NGROUP - 1).wait()


def kernel(x, table):
    return _emb_gather(x.astype(jnp.int32), table)
